# Initial kernel scaffold; baseline (speedup 1.0000x reference)
import jax
import jax.numpy as jnp
from jax.experimental import pallas as pl


def kernel(updates, mask):
    raise NotImplementedError("write your pallas kernel here")


# speedup vs baseline: 2.7457x; 2.7457x over previous
import jax
import jax.numpy as jnp
from jax import lax
from jax.experimental import pallas as pl
from jax.experimental.pallas import tpu as pltpu
from jax.experimental.pallas import tpu_sc as plsc

B, H, W, C = 4, 192, 192, 96
HW = H * W
P = (2 * H) * (2 * W)
CG = 4
NCG = C // CG
NTILES = 16
ROWS_PER_TILE = HW // NTILES
ELEMS_PER_TILE = ROWS_PER_TILE * CG
ACC_WORDS = P * CG
ACC_PER_TILE = ACC_WORDS // NTILES
CG_PER_CORE = NCG // 2
NVEC = ELEMS_PER_TILE // 16


def _body(upd_hbm, msk_hbm, zero_hbm, out_hbm, mbuf, ubuf, vbuf, ibuf, zbuf, acc):
    core = lax.axis_index("c")
    tile = lax.axis_index("s")

    lane = lax.iota(jnp.int32, 16)
    r_pat = lane // CG
    c_pat = lane - r_pat * CG
    inv_c = jnp.float32(1.0 / C)

    pltpu.sync_copy(zero_hbm, zbuf)

    def zero_acc():
        for j in range(ACC_PER_TILE // ELEMS_PER_TILE):
            pltpu.sync_copy(
                zbuf,
                acc.at[pl.ds(tile * ACC_PER_TILE + j * ELEMS_PER_TILE,
                             ELEMS_PER_TILE)])

    zero_acc()
    plsc.subcore_barrier()

    def item_step(it, _):
        b = it // CG_PER_CORE
        cg = core * CG_PER_CORE + (it - b * CG_PER_CORE)
        row0 = tile * ROWS_PER_TILE

        pltpu.sync_copy(msk_hbm.at[b, pl.ds(row0, ROWS_PER_TILE), cg, :], mbuf)
        pltpu.sync_copy(upd_hbm.at[b, pl.ds(row0, ROWS_PER_TILE), cg, :], ubuf)

        def compute(i, _):
            rows = r_pat + (16 // CG) * i
            m = plsc.load_gather(mbuf, [rows, c_pat])
            u = plsc.load_gather(ubuf, [rows, c_pat])
            q = (m.astype(jnp.float32) * inv_c).astype(jnp.int32)
            r = m - q * C
            q = jnp.where(r >= C, q + 1, q)
            q = jnp.where(r < 0, q - 1, q)
            ibuf[pl.ds(i * 16, 16)] = q * CG + c_pat
            vbuf[pl.ds(i * 16, 16)] = u
            return _
        lax.fori_loop(0, NVEC, compute, None)

        pltpu.sync_copy(vbuf, acc.at[ibuf], add=True)
        plsc.subcore_barrier()

        gitem = b * NCG + cg
        pltpu.sync_copy(
            acc.at[pl.ds(tile * ACC_PER_TILE, ACC_PER_TILE)],
            out_hbm.at[gitem, pl.ds(tile * ACC_PER_TILE, ACC_PER_TILE)])
        zero_acc()
        plsc.subcore_barrier()
        return _

    lax.fori_loop(0, B * CG_PER_CORE, item_step, None)


@jax.jit
def kernel(updates, mask):
    upd = updates.reshape(B, HW, NCG, CG)
    msk = mask.astype(jnp.int32).reshape(B, HW, NCG, CG)
    zero = jnp.zeros((ELEMS_PER_TILE,), jnp.float32)
    mesh = plsc.VectorSubcoreMesh(core_axis_name="c", subcore_axis_name="s")
    planar = pl.kernel(
        _body,
        out_type=jax.ShapeDtypeStruct((B * NCG, ACC_WORDS), jnp.float32),
        mesh=mesh,
        compiler_params=pltpu.CompilerParams(use_tc_tiling_on_sc=False, needs_layout_passes=False),
        scratch_types=[
            pltpu.VMEM((ROWS_PER_TILE, CG), jnp.int32),
            pltpu.VMEM((ROWS_PER_TILE, CG), jnp.float32),
            pltpu.VMEM((ELEMS_PER_TILE,), jnp.float32),
            pltpu.VMEM((ELEMS_PER_TILE,), jnp.int32),
            pltpu.VMEM((ELEMS_PER_TILE,), jnp.float32),
            pltpu.VMEM_SHARED((ACC_WORDS,), jnp.float32),
        ],
    )(upd, msk, zero)
    out = planar.reshape(B, NCG, P, CG).transpose(0, 2, 1, 3)
    return out.reshape(B, 2 * H, 2 * W, C)


# speedup vs baseline: 2.7477x; 1.0007x over previous
import jax
import jax.numpy as jnp
from jax import lax
from jax.experimental import pallas as pl
from jax.experimental.pallas import tpu as pltpu
from jax.experimental.pallas import tpu_sc as plsc

B, H, W, C = 4, 192, 192, 96
HW = H * W
P = (2 * H) * (2 * W)
CG = 4
NCG = C // CG
NTILES = 16
ROWS_PER_TILE = HW // NTILES
ELEMS_PER_TILE = ROWS_PER_TILE * CG
ACC_WORDS = P * CG
ACC_PER_TILE = ACC_WORDS // NTILES
CG_PER_CORE = NCG // 2
NVEC = ELEMS_PER_TILE // 16


def _body(upd_hbm, msk_hbm, zero_hbm, out_hbm, mbuf, ubuf, vbuf, ibuf, zbuf,
          acc):
    core = lax.axis_index("c")
    tile = lax.axis_index("s")

    lane = lax.iota(jnp.int32, 16)
    x_pat = lane // CG
    c_pat = lane - x_pat * CG
    inv_c = jnp.float32(1.0 / C)

    pltpu.sync_copy(zero_hbm, zbuf)

    def zero_acc():
        for j in range(ACC_PER_TILE // ELEMS_PER_TILE):
            pltpu.sync_copy(
                zbuf,
                acc.at[pl.ds(tile * ACC_PER_TILE + j * ELEMS_PER_TILE,
                             ELEMS_PER_TILE)])

    zero_acc()
    plsc.subcore_barrier()

    def item_step(it, _):
        b = it // CG_PER_CORE
        cg = core * CG_PER_CORE + (it - b * CG_PER_CORE)
        row0 = tile * ROWS_PER_TILE

        pltpu.sync_copy(msk_hbm.at[b, pl.ds(row0, ROWS_PER_TILE), cg, :],
                        mbuf)
        pltpu.sync_copy(upd_hbm.at[b, pl.ds(row0, ROWS_PER_TILE), cg, :],
                        ubuf)

        def compute(i, _):
            rows = x_pat + (16 // CG) * i
            m = plsc.load_gather(mbuf, [rows, c_pat])
            u = plsc.load_gather(ubuf, [rows, c_pat])
            q = (m.astype(jnp.float32) * inv_c).astype(jnp.int32)
            r = m - q * C
            q = jnp.where(r >= C, q + 1, q)
            q = jnp.where(r < 0, q - 1, q)
            ibuf[pl.ds(i * 16, 16)] = q * CG + c_pat
            vbuf[pl.ds(i * 16, 16)] = u
            return _
        lax.fori_loop(0, NVEC, compute, None, unroll=8)

        pltpu.sync_copy(vbuf, acc.at[ibuf], add=True)
        plsc.subcore_barrier()

        gitem = b * NCG + cg
        pltpu.sync_copy(
            acc.at[pl.ds(tile * ACC_PER_TILE, ACC_PER_TILE)],
            out_hbm.at[gitem, pl.ds(tile * ACC_PER_TILE, ACC_PER_TILE)])
        zero_acc()
        plsc.subcore_barrier()
        return _

    lax.fori_loop(0, B * CG_PER_CORE, item_step, None)


@jax.jit
def kernel(updates, mask):
    upd = updates.reshape(B, HW, NCG, CG)
    msk = mask.astype(jnp.int32).reshape(B, HW, NCG, CG)
    zero = jnp.zeros((ELEMS_PER_TILE,), jnp.float32)
    mesh = plsc.VectorSubcoreMesh(core_axis_name="c", subcore_axis_name="s")
    planar = pl.kernel(
        _body,
        out_type=jax.ShapeDtypeStruct((B * NCG, ACC_WORDS), jnp.float32),
        mesh=mesh,
        compiler_params=pltpu.CompilerParams(use_tc_tiling_on_sc=False,
                                             needs_layout_passes=False),
        scratch_types=[
            pltpu.VMEM((ROWS_PER_TILE, CG), jnp.int32),
            pltpu.VMEM((ROWS_PER_TILE, CG), jnp.float32),
            pltpu.VMEM((ELEMS_PER_TILE,), jnp.float32),
            pltpu.VMEM((ELEMS_PER_TILE,), jnp.int32),
            pltpu.VMEM((ELEMS_PER_TILE,), jnp.float32),
            pltpu.VMEM_SHARED((ACC_WORDS,), jnp.float32),
        ],
    )(upd, msk, zero)
    out = planar.reshape(B, NCG, P, CG).transpose(0, 2, 1, 3)
    return out.reshape(B, 2 * H, 2 * W, C)
